# baseline (speedup 1.0000x reference)
import functools

import jax
import jax.numpy as jnp
from jax import lax
from jax.experimental import pallas as pl
from jax.experimental.pallas import tpu as pltpu
from jax.experimental.pallas import tpu_sc as plsc

T = 2048
H = 768
I = 2048
E = 64
K = 2
R = T * K
TM = 256
NI = 2
IC = I // NI
CH = 128
NC = R // CH
NT = R // CH
NW = 32
NEG = -1e30


def _routing_body(x_ref, wg_ref, pos_ref, ends_ref, wf_ref, oh_ref):
    x = x_ref[...]
    wg = wg_ref[...]
    logits = lax.dot_general(x, wg, (((1,), (1,)), ((), ())),
                             preferred_element_type=jnp.float32)

    e_iota = lax.broadcasted_iota(jnp.int32, (T, E), 1)
    m0 = jnp.max(logits, axis=1, keepdims=True)
    e0 = jnp.min(jnp.where(logits == m0, e_iota, E), axis=1, keepdims=True)
    oh0 = e_iota == e0
    masked = jnp.where(oh0, NEG, logits)
    m1 = jnp.max(masked, axis=1, keepdims=True)
    e1 = jnp.min(jnp.where(masked == m1, e_iota, E), axis=1, keepdims=True)
    oh1 = e_iota == e1

    z = jnp.exp(m1 - m0)
    w0 = 1.0 / (1.0 + z)
    w1 = 1.0 - w0

    oh = jnp.concatenate([oh0, oh1], axis=0).astype(jnp.float32)
    oh_ref[...] = oh
    counts = jnp.sum(oh, axis=0, keepdims=True)
    me = (lax.broadcasted_iota(jnp.int32, (E, E), 0)
          <= lax.broadcasted_iota(jnp.int32, (E, E), 1)).astype(jnp.float32)
    ends = lax.dot_general(counts, me, (((1,), (0,)), ((), ())),
                           preferred_element_type=jnp.float32)
    offsets = ends - counts
    ml = (lax.broadcasted_iota(jnp.int32, (CH, CH), 1)
          <= lax.broadcasted_iota(jnp.int32, (CH, CH), 0)).astype(jnp.float32)

    def rchunk(c, carry):
        r0 = pl.multiple_of(c * CH, CH)
        ohc = oh_ref[pl.ds(r0, CH), :]
        csc = lax.dot_general(ml, ohc, (((1,), (0,)), ((), ())),
                              preferred_element_type=jnp.float32) + carry
        rank = jnp.sum(csc * ohc, axis=1, keepdims=True) - 1.0
        offs_r = jnp.sum(ohc * offsets, axis=1, keepdims=True)
        pos_ref[pl.ds(r0, CH), :] = (rank + offs_r).astype(jnp.int32)
        return carry + jnp.sum(ohc, axis=0, keepdims=True)

    lax.fori_loop(0, NC, rchunk, jnp.zeros((1, E), jnp.float32))
    ends_ref[...] = ends.astype(jnp.int32)
    wf_ref[...] = jnp.concatenate([w0, w1], axis=0)


def _routing(x2, wg):
    return pl.pallas_call(
        _routing_body,
        out_shape=(
            jax.ShapeDtypeStruct((R, 1), jnp.int32),
            jax.ShapeDtypeStruct((1, E), jnp.int32),
            jax.ShapeDtypeStruct((R, 1), jnp.float32),
        ),
        scratch_shapes=[pltpu.VMEM((R, E), jnp.float32)],
    )(x2, wg)


def _sc_scatter_ids(pos, tokflat, wflat):
    bpw = R // NW
    mesh = plsc.VectorSubcoreMesh(core_axis_name="c", subcore_axis_name="s")

    @functools.partial(
        pl.kernel,
        mesh=mesh,
        out_type=(
            jax.ShapeDtypeStruct((R,), jnp.int32),
            jax.ShapeDtypeStruct((R,), jnp.float32),
        ),
        scratch_types=[
            pltpu.VMEM((bpw,), jnp.int32),
            pltpu.VMEM((bpw,), jnp.int32),
            pltpu.VMEM((bpw,), jnp.float32),
            pltpu.SemaphoreType.DMA,
        ],
    )
    def k(pos_hbm, tok_hbm, wf_hbm, st_hbm, sw_hbm, idx_v, tok_v, wf_v, sem):
        wid = lax.axis_index("s") * 2 + lax.axis_index("c")
        base = pl.multiple_of(wid * bpw, bpw)
        pltpu.sync_copy(pos_hbm.at[pl.ds(base, bpw)], idx_v)
        pltpu.sync_copy(tok_hbm.at[pl.ds(base, bpw)], tok_v)
        pltpu.sync_copy(wf_hbm.at[pl.ds(base, bpw)], wf_v)
        pltpu.async_copy(tok_v, st_hbm.at[idx_v], sem).wait()
        pltpu.async_copy(wf_v, sw_hbm.at[idx_v], sem).wait()

    return k(pos, tokflat, wflat)


def _sc_gather(x2, sorted_token):
    bpw = R // NW
    mesh = plsc.VectorSubcoreMesh(core_axis_name="c", subcore_axis_name="s")

    @functools.partial(
        pl.kernel,
        mesh=mesh,
        out_type=jax.ShapeDtypeStruct((R, H), jnp.float32),
        scratch_types=[
            pltpu.VMEM((bpw,), jnp.int32),
            pltpu.VMEM((bpw, H), jnp.float32),
            pltpu.SemaphoreType.DMA,
        ],
    )
    def k(x_hbm, st_hbm, out_hbm, idx_v, rows_v, sem):
        wid = lax.axis_index("s") * 2 + lax.axis_index("c")
        base = pl.multiple_of(wid * bpw, bpw)
        pltpu.sync_copy(st_hbm.at[pl.ds(base, bpw)], idx_v)
        pltpu.async_copy(x_hbm.at[idx_v], rows_v, sem).wait()
        pltpu.sync_copy(rows_v, out_hbm.at[pl.ds(base, bpw)])

    return k(x2, sorted_token)


def _gmm_body(offs_ref, xs_ref, w1_ref, w3_ref, w2_ref, sw_ref, ys_ref):
    e = pl.program_id(0)
    ic = pl.program_id(1)

    @pl.when((e == 0) & (ic == 0))
    def _zero():
        ys_ref[...] = jnp.zeros_like(ys_ref)

    lo = offs_ref[e]
    hi = offs_ref[e + 1]
    t_lo = lo // TM
    nt = (hi + TM - 1) // TM - t_lo

    w1 = w1_ref[0]
    w3 = w3_ref[0]
    w2 = w2_ref[0]

    def tile(j, _):
        r0 = pl.multiple_of((t_lo + j) * TM, TM)
        xs = xs_ref[pl.ds(r0, TM), :]
        h1 = lax.dot_general(xs, w1, (((1,), (1,)), ((), ())),
                             preferred_element_type=jnp.float32)
        h3 = lax.dot_general(xs, w3, (((1,), (1,)), ((), ())),
                             preferred_element_type=jnp.float32)
        g = h1 * lax.logistic(h1) * h3
        y = lax.dot_general(g, w2, (((1,), (1,)), ((), ())),
                            preferred_element_type=jnp.float32)
        rows = lax.broadcasted_iota(jnp.int32, (TM, 1), 0) + r0
        w = jnp.where((rows >= lo) & (rows < hi), sw_ref[pl.ds(r0, TM), :], 0.0)
        ys_ref[pl.ds(r0, TM), :] += y * w
        return 0

    lax.fori_loop(0, nt, tile, 0)


def _gmm(offs, xs, w1, w3, w2, sw):
    grid_spec = pltpu.PrefetchScalarGridSpec(
        num_scalar_prefetch=1,
        grid=(E, NI),
        in_specs=[
            pl.BlockSpec((R, H), lambda e, i, offs: (0, 0)),
            pl.BlockSpec((1, IC, H), lambda e, i, offs: (e, i, 0)),
            pl.BlockSpec((1, IC, H), lambda e, i, offs: (e, i, 0)),
            pl.BlockSpec((1, H, IC), lambda e, i, offs: (e, 0, i)),
            pl.BlockSpec((R, 1), lambda e, i, offs: (0, 0)),
        ],
        out_specs=pl.BlockSpec((R, H), lambda e, i, offs: (0, 0)),
    )
    return pl.pallas_call(
        _gmm_body,
        grid_spec=grid_spec,
        out_shape=jax.ShapeDtypeStruct((R, H), jnp.float32),
        compiler_params=pltpu.CompilerParams(vmem_limit_bytes=63 * 1024 * 1024),
    )(offs, xs, w1, w3, w2, sw)


def _sc_combine(ys, p0, p1):
    tpw = T // NW
    mesh = plsc.VectorSubcoreMesh(core_axis_name="c", subcore_axis_name="s")

    @functools.partial(
        pl.kernel,
        mesh=mesh,
        out_type=jax.ShapeDtypeStruct((T, H), jnp.float32),
        scratch_types=[
            pltpu.VMEM((tpw,), jnp.int32),
            pltpu.VMEM((tpw,), jnp.int32),
            pltpu.VMEM((tpw, H), jnp.float32),
            pltpu.VMEM((tpw, H), jnp.float32),
            pltpu.SemaphoreType.DMA,
        ],
    )
    def k(ys_hbm, p0_hbm, p1_hbm, out_hbm, i0_v, i1_v, a_v, b_v, sem):
        wid = lax.axis_index("s") * 2 + lax.axis_index("c")
        base = wid * tpw
        pltpu.sync_copy(p0_hbm.at[pl.ds(base, tpw)], i0_v)
        pltpu.sync_copy(p1_hbm.at[pl.ds(base, tpw)], i1_v)
        ca = pltpu.async_copy(ys_hbm.at[i0_v], a_v, sem)
        cb = pltpu.async_copy(ys_hbm.at[i1_v], b_v, sem)
        ca.wait()
        cb.wait()

        def row(i, _):
            def lane(c, _2):
                sl = pl.ds(c * 16, 16)
                a_v[i, sl] = a_v[i, sl] + b_v[i, sl]
                return 0
            lax.fori_loop(0, H // 16, lane, 0)
            return 0

        lax.fori_loop(0, tpw, row, 0)
        pltpu.sync_copy(a_v, out_hbm.at[pl.ds(base, tpw)])

    return k(ys, p0, p1)


def kernel(x, Wg, W1, W2, W3):
    b, s, h = x.shape
    x2 = x.reshape(T, H)
    pos, ends, wflat = _routing(x2, Wg)
    offs = jnp.concatenate([jnp.zeros((1,), jnp.int32), ends.reshape(E)])
    posf = pos.reshape(R)
    tokflat = jnp.arange(R, dtype=jnp.int32) % T
    st, sw = _sc_scatter_ids(posf, tokflat, wflat.reshape(R))
    xs = _sc_gather(x2, st)
    ys = _gmm(offs, xs, W1, W3, W2, sw.reshape(R, 1))
    out = _sc_combine(ys, posf[:T], posf[T:])
    return out.reshape(b, s, h)


# speedup vs baseline: 1.0490x; 1.0490x over previous
import functools

import jax
import jax.numpy as jnp
from jax import lax
from jax.experimental import pallas as pl
from jax.experimental.pallas import tpu as pltpu
from jax.experimental.pallas import tpu_sc as plsc

T = 2048
H = 768
I = 2048
E = 64
K = 2
R = T * K
TM = 256
NI = 2
IC = I // NI
CH = 128
NC = R // CH
NT = R // CH
NW = 32
NEG = -1e30


def _routing_body(x_ref, wg_ref, pos_ref, ends_ref, wf_ref, oh_ref):
    x = x_ref[...]
    wg = wg_ref[...]
    logits = lax.dot_general(x, wg, (((1,), (1,)), ((), ())),
                             preferred_element_type=jnp.float32)

    e_iota = lax.broadcasted_iota(jnp.int32, (T, E), 1)
    m0 = jnp.max(logits, axis=1, keepdims=True)
    e0 = jnp.min(jnp.where(logits == m0, e_iota, E), axis=1, keepdims=True)
    oh0 = e_iota == e0
    masked = jnp.where(oh0, NEG, logits)
    m1 = jnp.max(masked, axis=1, keepdims=True)
    e1 = jnp.min(jnp.where(masked == m1, e_iota, E), axis=1, keepdims=True)
    oh1 = e_iota == e1

    z = jnp.exp(m1 - m0)
    w0 = 1.0 / (1.0 + z)
    w1 = 1.0 - w0

    oh = jnp.concatenate([oh0, oh1], axis=0).astype(jnp.float32)
    oh_ref[...] = oh
    counts = jnp.sum(oh, axis=0, keepdims=True)
    me = (lax.broadcasted_iota(jnp.int32, (E, E), 0)
          <= lax.broadcasted_iota(jnp.int32, (E, E), 1)).astype(jnp.float32)
    ends = lax.dot_general(counts, me, (((1,), (0,)), ((), ())),
                           preferred_element_type=jnp.float32)
    offsets = ends - counts
    ml = (lax.broadcasted_iota(jnp.int32, (CH, CH), 1)
          <= lax.broadcasted_iota(jnp.int32, (CH, CH), 0)).astype(jnp.float32)

    def rchunk(c, carry):
        r0 = pl.multiple_of(c * CH, CH)
        ohc = oh_ref[pl.ds(r0, CH), :]
        csc = lax.dot_general(ml, ohc, (((1,), (0,)), ((), ())),
                              preferred_element_type=jnp.float32) + carry
        rank = jnp.sum(csc * ohc, axis=1, keepdims=True) - 1.0
        offs_r = jnp.sum(ohc * offsets, axis=1, keepdims=True)
        pos_ref[pl.ds(r0, CH), :] = (rank + offs_r).astype(jnp.int32)
        return carry + jnp.sum(ohc, axis=0, keepdims=True)

    lax.fori_loop(0, NC, rchunk, jnp.zeros((1, E), jnp.float32))
    ends_ref[...] = ends.astype(jnp.int32)
    wf_ref[...] = jnp.concatenate([w0, w1], axis=0)


def _routing(x2, wg):
    return pl.pallas_call(
        _routing_body,
        out_shape=(
            jax.ShapeDtypeStruct((R, 1), jnp.int32),
            jax.ShapeDtypeStruct((1, E), jnp.int32),
            jax.ShapeDtypeStruct((R, 1), jnp.float32),
        ),
        scratch_shapes=[pltpu.VMEM((R, E), jnp.float32)],
    )(x2, wg)


def _sc_dispatch(x2, pos, wflat):
    bpw = R // NW
    mesh = plsc.VectorSubcoreMesh(core_axis_name="c", subcore_axis_name="s")

    @functools.partial(
        pl.kernel,
        mesh=mesh,
        out_type=(
            jax.ShapeDtypeStruct((R, H), jnp.float32),
            jax.ShapeDtypeStruct((R,), jnp.float32),
        ),
        scratch_types=[
            pltpu.VMEM((bpw,), jnp.int32),
            pltpu.VMEM((bpw, H), jnp.float32),
            pltpu.VMEM((bpw,), jnp.float32),
            pltpu.SemaphoreType.DMA,
        ],
    )
    def k(x_hbm, pos_hbm, wf_hbm, xs_hbm, sw_hbm, idx_v, rows_v, wf_v, sem):
        wid = lax.axis_index("s") * 2 + lax.axis_index("c")
        base = pl.multiple_of(wid * bpw, bpw)
        tb = pl.multiple_of(jnp.bitwise_and(base, T - 1), bpw)
        ca = pltpu.async_copy(x_hbm.at[pl.ds(tb, bpw)], rows_v, sem)
        pltpu.sync_copy(pos_hbm.at[pl.ds(base, bpw)], idx_v)
        pltpu.sync_copy(wf_hbm.at[pl.ds(base, bpw)], wf_v)
        ca.wait()
        cb = pltpu.async_copy(rows_v, xs_hbm.at[idx_v], sem)
        cc = pltpu.async_copy(wf_v, sw_hbm.at[idx_v], sem)
        cb.wait()
        cc.wait()

    return k(x2, pos, wflat)


def _gmm_body(offs_ref, xs_ref, w1_ref, w3_ref, w2_ref, sw_ref, ys_ref):
    e = pl.program_id(0)
    ic = pl.program_id(1)

    @pl.when((e == 0) & (ic == 0))
    def _zero():
        ys_ref[...] = jnp.zeros_like(ys_ref)

    lo = offs_ref[e]
    hi = offs_ref[e + 1]
    t_lo = lo // TM
    nt = (hi + TM - 1) // TM - t_lo

    w1 = w1_ref[0]
    w3 = w3_ref[0]
    w2 = w2_ref[0]

    def tile(j, _):
        r0 = pl.multiple_of((t_lo + j) * TM, TM)
        xs = xs_ref[pl.ds(r0, TM), :]
        h1 = lax.dot_general(xs, w1, (((1,), (1,)), ((), ())),
                             preferred_element_type=jnp.float32)
        h3 = lax.dot_general(xs, w3, (((1,), (1,)), ((), ())),
                             preferred_element_type=jnp.float32)
        g = h1 * lax.logistic(h1) * h3
        y = lax.dot_general(g, w2, (((1,), (1,)), ((), ())),
                            preferred_element_type=jnp.float32)
        rows = lax.broadcasted_iota(jnp.int32, (TM, 1), 0) + r0
        w = jnp.where((rows >= lo) & (rows < hi), sw_ref[pl.ds(r0, TM), :], 0.0)
        ys_ref[pl.ds(r0, TM), :] += y * w
        return 0

    lax.fori_loop(0, nt, tile, 0)


def _gmm(offs, xs, w1, w3, w2, sw):
    grid_spec = pltpu.PrefetchScalarGridSpec(
        num_scalar_prefetch=1,
        grid=(E, NI),
        in_specs=[
            pl.BlockSpec((R, H), lambda e, i, offs: (0, 0)),
            pl.BlockSpec((1, IC, H), lambda e, i, offs: (e, i, 0)),
            pl.BlockSpec((1, IC, H), lambda e, i, offs: (e, i, 0)),
            pl.BlockSpec((1, H, IC), lambda e, i, offs: (e, 0, i)),
            pl.BlockSpec((R, 1), lambda e, i, offs: (0, 0)),
        ],
        out_specs=pl.BlockSpec((R, H), lambda e, i, offs: (0, 0)),
    )
    return pl.pallas_call(
        _gmm_body,
        grid_spec=grid_spec,
        out_shape=jax.ShapeDtypeStruct((R, H), jnp.float32),
        compiler_params=pltpu.CompilerParams(vmem_limit_bytes=63 * 1024 * 1024),
    )(offs, xs, w1, w3, w2, sw)


def _sc_combine(ys, p0, p1):
    tpw = T // NW
    mesh = plsc.VectorSubcoreMesh(core_axis_name="c", subcore_axis_name="s")

    @functools.partial(
        pl.kernel,
        mesh=mesh,
        out_type=jax.ShapeDtypeStruct((T, H), jnp.float32),
        scratch_types=[
            pltpu.VMEM((tpw,), jnp.int32),
            pltpu.VMEM((tpw,), jnp.int32),
            pltpu.VMEM((tpw, H), jnp.float32),
            pltpu.VMEM((tpw, H), jnp.float32),
            pltpu.SemaphoreType.DMA,
        ],
    )
    def k(ys_hbm, p0_hbm, p1_hbm, out_hbm, i0_v, i1_v, a_v, b_v, sem):
        wid = lax.axis_index("s") * 2 + lax.axis_index("c")
        base = wid * tpw
        pltpu.sync_copy(p0_hbm.at[pl.ds(base, tpw)], i0_v)
        pltpu.sync_copy(p1_hbm.at[pl.ds(base, tpw)], i1_v)
        ca = pltpu.async_copy(ys_hbm.at[i0_v], a_v, sem)
        cb = pltpu.async_copy(ys_hbm.at[i1_v], b_v, sem)
        ca.wait()
        cb.wait()

        def row(i, _):
            for c in range(H // 16):
                sl = pl.ds(c * 16, 16)
                a_v[i, sl] = a_v[i, sl] + b_v[i, sl]
            return 0

        lax.fori_loop(0, tpw, row, 0)
        pltpu.sync_copy(a_v, out_hbm.at[pl.ds(base, tpw)])

    return k(ys, p0, p1)


def kernel(x, Wg, W1, W2, W3):
    b, s, h = x.shape
    x2 = x.reshape(T, H)
    pos, ends, wflat = _routing(x2, Wg)
    offs = jnp.concatenate([jnp.zeros((1,), jnp.int32), ends.reshape(E)])
    posf = pos.reshape(R)
    xs, sw = _sc_dispatch(x2, posf, wflat.reshape(R))
    ys = _gmm(offs, xs, W1, W3, W2, sw.reshape(R, 1))
    out = _sc_combine(ys, posf[:T], posf[T:])
    return out.reshape(b, s, h)


# speedup vs baseline: 1.0538x; 1.0046x over previous
import functools

import jax
import jax.numpy as jnp
from jax import lax
from jax.experimental import pallas as pl
from jax.experimental.pallas import tpu as pltpu
from jax.experimental.pallas import tpu_sc as plsc

T = 2048
H = 768
I = 2048
E = 64
K = 2
R = T * K
TM = 256
NI = 2
IC = I // NI
CH = 256
NC = R // CH
NT = R // CH
NW = 32
NEG = -1e30


def _routing_body(x_ref, wg_ref, pos_ref, ends_ref, wf_ref, oh_ref):
    x = x_ref[...]
    wg = wg_ref[...]
    logits = lax.dot_general(x, wg, (((1,), (1,)), ((), ())),
                             preferred_element_type=jnp.float32)

    e_iota = lax.broadcasted_iota(jnp.int32, (T, E), 1)
    m0 = jnp.max(logits, axis=1, keepdims=True)
    e0 = jnp.min(jnp.where(logits == m0, e_iota, E), axis=1, keepdims=True)
    oh0 = e_iota == e0
    masked = jnp.where(oh0, NEG, logits)
    m1 = jnp.max(masked, axis=1, keepdims=True)
    e1 = jnp.min(jnp.where(masked == m1, e_iota, E), axis=1, keepdims=True)
    oh1 = e_iota == e1

    z = jnp.exp(m1 - m0)
    w0 = 1.0 / (1.0 + z)
    w1 = 1.0 - w0

    oh = jnp.concatenate([oh0, oh1], axis=0).astype(jnp.float32)
    oh_ref[...] = oh
    counts = jnp.sum(oh, axis=0, keepdims=True)
    me = (lax.broadcasted_iota(jnp.int32, (E, E), 0)
          <= lax.broadcasted_iota(jnp.int32, (E, E), 1)).astype(jnp.float32)
    ends = lax.dot_general(counts, me, (((1,), (0,)), ((), ())),
                           preferred_element_type=jnp.float32)
    offsets = ends - counts
    ml = (lax.broadcasted_iota(jnp.int32, (CH, CH), 1)
          <= lax.broadcasted_iota(jnp.int32, (CH, CH), 0)).astype(jnp.float32)

    def rchunk(c, carry):
        r0 = pl.multiple_of(c * CH, CH)
        ohc = oh_ref[pl.ds(r0, CH), :]
        csc = lax.dot_general(ml, ohc, (((1,), (0,)), ((), ())),
                              preferred_element_type=jnp.float32) + carry
        rank = jnp.sum(csc * ohc, axis=1, keepdims=True) - 1.0
        offs_r = jnp.sum(ohc * offsets, axis=1, keepdims=True)
        pos_ref[pl.ds(r0, CH), :] = (rank + offs_r).astype(jnp.int32)
        return carry + jnp.sum(ohc, axis=0, keepdims=True)

    lax.fori_loop(0, NC, rchunk, jnp.zeros((1, E), jnp.float32))
    ends_ref[...] = ends.astype(jnp.int32)
    wf_ref[...] = jnp.concatenate([w0, w1], axis=0)


def _routing(x2, wg):
    return pl.pallas_call(
        _routing_body,
        out_shape=(
            jax.ShapeDtypeStruct((R, 1), jnp.int32),
            jax.ShapeDtypeStruct((1, E), jnp.int32),
            jax.ShapeDtypeStruct((R, 1), jnp.float32),
        ),
        scratch_shapes=[pltpu.VMEM((R, E), jnp.float32)],
    )(x2, wg)


def _sc_dispatch(x2, pos, wflat):
    bpw = R // NW
    mesh = plsc.VectorSubcoreMesh(core_axis_name="c", subcore_axis_name="s")

    @functools.partial(
        pl.kernel,
        mesh=mesh,
        out_type=(
            jax.ShapeDtypeStruct((R, H), jnp.float32),
            jax.ShapeDtypeStruct((R,), jnp.float32),
        ),
        scratch_types=[
            pltpu.VMEM((bpw,), jnp.int32),
            pltpu.VMEM((bpw, H), jnp.float32),
            pltpu.VMEM((bpw,), jnp.float32),
            pltpu.SemaphoreType.DMA,
        ],
    )
    def k(x_hbm, pos_hbm, wf_hbm, xs_hbm, sw_hbm, idx_v, rows_v, wf_v, sem):
        wid = lax.axis_index("s") * 2 + lax.axis_index("c")
        base = pl.multiple_of(wid * bpw, bpw)
        tb = pl.multiple_of(jnp.bitwise_and(base, T - 1), bpw)
        ca = pltpu.async_copy(x_hbm.at[pl.ds(tb, bpw)], rows_v, sem)
        pltpu.sync_copy(pos_hbm.at[pl.ds(base, bpw)], idx_v)
        pltpu.sync_copy(wf_hbm.at[pl.ds(base, bpw)], wf_v)
        ca.wait()
        cb = pltpu.async_copy(rows_v, xs_hbm.at[idx_v], sem)
        cc = pltpu.async_copy(wf_v, sw_hbm.at[idx_v], sem)
        cb.wait()
        cc.wait()

    return k(x2, pos, wflat)


def _gmm_body(offs_ref, xs_ref, w1_ref, w3_ref, w2_ref, sw_ref, ys_ref):
    e = pl.program_id(0)
    ic = pl.program_id(1)

    @pl.when((e == 0) & (ic == 0))
    def _zero():
        ys_ref[...] = jnp.zeros_like(ys_ref)

    lo = offs_ref[e]
    hi = offs_ref[e + 1]
    t_lo = lo // TM
    nt = (hi + TM - 1) // TM - t_lo

    w1 = w1_ref[0]
    w3 = w3_ref[0]
    w2 = w2_ref[0]

    def tile(j, _):
        r0 = pl.multiple_of((t_lo + j) * TM, TM)
        xs = xs_ref[pl.ds(r0, TM), :]
        h1 = lax.dot_general(xs, w1, (((1,), (1,)), ((), ())),
                             preferred_element_type=jnp.float32)
        h3 = lax.dot_general(xs, w3, (((1,), (1,)), ((), ())),
                             preferred_element_type=jnp.float32)
        g = h1 * lax.logistic(h1) * h3
        y = lax.dot_general(g, w2, (((1,), (1,)), ((), ())),
                            preferred_element_type=jnp.float32)
        rows = lax.broadcasted_iota(jnp.int32, (TM, 1), 0) + r0
        w = jnp.where((rows >= lo) & (rows < hi), sw_ref[pl.ds(r0, TM), :], 0.0)
        ys_ref[pl.ds(r0, TM), :] += y * w
        return 0

    lax.fori_loop(0, nt, tile, 0)


def _gmm(offs, xs, w1, w3, w2, sw):
    grid_spec = pltpu.PrefetchScalarGridSpec(
        num_scalar_prefetch=1,
        grid=(E, NI),
        in_specs=[
            pl.BlockSpec((R, H), lambda e, i, offs: (0, 0)),
            pl.BlockSpec((1, IC, H), lambda e, i, offs: (e, i, 0)),
            pl.BlockSpec((1, IC, H), lambda e, i, offs: (e, i, 0)),
            pl.BlockSpec((1, H, IC), lambda e, i, offs: (e, 0, i)),
            pl.BlockSpec((R, 1), lambda e, i, offs: (0, 0)),
        ],
        out_specs=pl.BlockSpec((R, H), lambda e, i, offs: (0, 0)),
    )
    return pl.pallas_call(
        _gmm_body,
        grid_spec=grid_spec,
        out_shape=jax.ShapeDtypeStruct((R, H), jnp.float32),
        compiler_params=pltpu.CompilerParams(vmem_limit_bytes=63 * 1024 * 1024),
    )(offs, xs, w1, w3, w2, sw)


def _sc_combine(ys, p0, p1):
    tpw = T // NW
    mesh = plsc.VectorSubcoreMesh(core_axis_name="c", subcore_axis_name="s")

    @functools.partial(
        pl.kernel,
        mesh=mesh,
        out_type=jax.ShapeDtypeStruct((T, H), jnp.float32),
        scratch_types=[
            pltpu.VMEM((tpw,), jnp.int32),
            pltpu.VMEM((tpw,), jnp.int32),
            pltpu.VMEM((tpw, H), jnp.float32),
            pltpu.VMEM((tpw, H), jnp.float32),
            pltpu.SemaphoreType.DMA,
        ],
    )
    def k(ys_hbm, p0_hbm, p1_hbm, out_hbm, i0_v, i1_v, a_v, b_v, sem):
        wid = lax.axis_index("s") * 2 + lax.axis_index("c")
        base = wid * tpw
        pltpu.sync_copy(p0_hbm.at[pl.ds(base, tpw)], i0_v)
        pltpu.sync_copy(p1_hbm.at[pl.ds(base, tpw)], i1_v)
        ca = pltpu.async_copy(ys_hbm.at[i0_v], a_v, sem)
        cb = pltpu.async_copy(ys_hbm.at[i1_v], b_v, sem)
        ca.wait()
        cb.wait()

        def row(i, _):
            for c in range(H // 16):
                sl = pl.ds(c * 16, 16)
                a_v[i, sl] = a_v[i, sl] + b_v[i, sl]
            return 0

        lax.fori_loop(0, tpw, row, 0)
        pltpu.sync_copy(a_v, out_hbm.at[pl.ds(base, tpw)])

    return k(ys, p0, p1)


def kernel(x, Wg, W1, W2, W3):
    b, s, h = x.shape
    x2 = x.reshape(T, H)
    pos, ends, wflat = _routing(x2, Wg)
    offs = jnp.concatenate([jnp.zeros((1,), jnp.int32), ends.reshape(E)])
    posf = pos.reshape(R)
    xs, sw = _sc_dispatch(x2, posf, wflat.reshape(R))
    ys = _gmm(offs, xs, W1, W3, W2, sw.reshape(R, 1))
    out = _sc_combine(ys, posf[:T], posf[T:])
    return out.reshape(b, s, h)


# speedup vs baseline: 1.0553x; 1.0014x over previous
import functools

import jax
import jax.numpy as jnp
from jax import lax
from jax.experimental import pallas as pl
from jax.experimental.pallas import tpu as pltpu
from jax.experimental.pallas import tpu_sc as plsc

T = 2048
H = 768
I = 2048
E = 64
K = 2
R = T * K
TM = 256
NI = 2
IC = I // NI
CH = 512
NC = R // CH
NT = R // CH
NW = 32
NEG = -1e30


def _routing_body(x_ref, wg_ref, pos_ref, ends_ref, wf_ref, oh_ref):
    x = x_ref[...]
    wg = wg_ref[...]
    logits = lax.dot_general(x, wg, (((1,), (1,)), ((), ())),
                             preferred_element_type=jnp.float32)

    e_iota = lax.broadcasted_iota(jnp.int32, (T, E), 1)
    m0 = jnp.max(logits, axis=1, keepdims=True)
    e0 = jnp.min(jnp.where(logits == m0, e_iota, E), axis=1, keepdims=True)
    oh0 = e_iota == e0
    masked = jnp.where(oh0, NEG, logits)
    m1 = jnp.max(masked, axis=1, keepdims=True)
    e1 = jnp.min(jnp.where(masked == m1, e_iota, E), axis=1, keepdims=True)
    oh1 = e_iota == e1

    z = jnp.exp(m1 - m0)
    w0 = 1.0 / (1.0 + z)
    w1 = 1.0 - w0

    oh = jnp.concatenate([oh0, oh1], axis=0).astype(jnp.float32)
    oh_ref[...] = oh
    counts = jnp.sum(oh, axis=0, keepdims=True)
    me = (lax.broadcasted_iota(jnp.int32, (E, E), 0)
          <= lax.broadcasted_iota(jnp.int32, (E, E), 1)).astype(jnp.float32)
    ends = lax.dot_general(counts, me, (((1,), (0,)), ((), ())),
                           preferred_element_type=jnp.float32)
    offsets = ends - counts
    ml = (lax.broadcasted_iota(jnp.int32, (CH, CH), 1)
          <= lax.broadcasted_iota(jnp.int32, (CH, CH), 0)).astype(jnp.float32)

    def rchunk(c, carry):
        r0 = pl.multiple_of(c * CH, CH)
        ohc = oh_ref[pl.ds(r0, CH), :]
        csc = lax.dot_general(ml, ohc, (((1,), (0,)), ((), ())),
                              preferred_element_type=jnp.float32) + carry
        rank = jnp.sum(csc * ohc, axis=1, keepdims=True) - 1.0
        offs_r = jnp.sum(ohc * offsets, axis=1, keepdims=True)
        pos_ref[pl.ds(r0, CH), :] = (rank + offs_r).astype(jnp.int32)
        return carry + jnp.sum(ohc, axis=0, keepdims=True)

    lax.fori_loop(0, NC, rchunk, jnp.zeros((1, E), jnp.float32))
    ends_ref[...] = ends.astype(jnp.int32)
    wf_ref[...] = jnp.concatenate([w0, w1], axis=0)


def _routing(x2, wg):
    return pl.pallas_call(
        _routing_body,
        out_shape=(
            jax.ShapeDtypeStruct((R, 1), jnp.int32),
            jax.ShapeDtypeStruct((1, E), jnp.int32),
            jax.ShapeDtypeStruct((R, 1), jnp.float32),
        ),
        scratch_shapes=[pltpu.VMEM((R, E), jnp.float32)],
    )(x2, wg)


def _sc_dispatch(x2, pos, wflat):
    bpw = R // NW
    mesh = plsc.VectorSubcoreMesh(core_axis_name="c", subcore_axis_name="s")

    @functools.partial(
        pl.kernel,
        mesh=mesh,
        out_type=(
            jax.ShapeDtypeStruct((R, H), jnp.float32),
            jax.ShapeDtypeStruct((R,), jnp.float32),
        ),
        scratch_types=[
            pltpu.VMEM((bpw,), jnp.int32),
            pltpu.VMEM((bpw, H), jnp.float32),
            pltpu.VMEM((bpw,), jnp.float32),
            pltpu.SemaphoreType.DMA,
        ],
    )
    def k(x_hbm, pos_hbm, wf_hbm, xs_hbm, sw_hbm, idx_v, rows_v, wf_v, sem):
        wid = lax.axis_index("s") * 2 + lax.axis_index("c")
        base = pl.multiple_of(wid * bpw, bpw)
        tb = pl.multiple_of(jnp.bitwise_and(base, T - 1), bpw)
        ca = pltpu.async_copy(x_hbm.at[pl.ds(tb, bpw)], rows_v, sem)
        pltpu.sync_copy(pos_hbm.at[pl.ds(base, bpw)], idx_v)
        pltpu.sync_copy(wf_hbm.at[pl.ds(base, bpw)], wf_v)
        ca.wait()
        cb = pltpu.async_copy(rows_v, xs_hbm.at[idx_v], sem)
        cc = pltpu.async_copy(wf_v, sw_hbm.at[idx_v], sem)
        cb.wait()
        cc.wait()

    return k(x2, pos, wflat)


def _gmm_body(offs_ref, xs_ref, w1_ref, w3_ref, w2_ref, sw_ref, ys_ref):
    e = pl.program_id(0)
    ic = pl.program_id(1)

    @pl.when((e == 0) & (ic == 0))
    def _zero():
        ys_ref[...] = jnp.zeros_like(ys_ref)

    lo = offs_ref[e]
    hi = offs_ref[e + 1]
    t_lo = lo // TM
    nt = (hi + TM - 1) // TM - t_lo

    w1 = w1_ref[0]
    w3 = w3_ref[0]
    w2 = w2_ref[0]

    def tile(j, _):
        r0 = pl.multiple_of((t_lo + j) * TM, TM)
        xs = xs_ref[pl.ds(r0, TM), :]
        h1 = lax.dot_general(xs, w1, (((1,), (1,)), ((), ())),
                             preferred_element_type=jnp.float32)
        h3 = lax.dot_general(xs, w3, (((1,), (1,)), ((), ())),
                             preferred_element_type=jnp.float32)
        g = h1 * lax.logistic(h1) * h3
        y = lax.dot_general(g, w2, (((1,), (1,)), ((), ())),
                            preferred_element_type=jnp.float32)
        rows = lax.broadcasted_iota(jnp.int32, (TM, 1), 0) + r0
        w = jnp.where((rows >= lo) & (rows < hi), sw_ref[pl.ds(r0, TM), :], 0.0)
        ys_ref[pl.ds(r0, TM), :] += y * w
        return 0

    lax.fori_loop(0, nt, tile, 0)


def _gmm(offs, xs, w1, w3, w2, sw):
    grid_spec = pltpu.PrefetchScalarGridSpec(
        num_scalar_prefetch=1,
        grid=(E, NI),
        in_specs=[
            pl.BlockSpec((R, H), lambda e, i, offs: (0, 0)),
            pl.BlockSpec((1, IC, H), lambda e, i, offs: (e, i, 0)),
            pl.BlockSpec((1, IC, H), lambda e, i, offs: (e, i, 0)),
            pl.BlockSpec((1, H, IC), lambda e, i, offs: (e, 0, i)),
            pl.BlockSpec((R, 1), lambda e, i, offs: (0, 0)),
        ],
        out_specs=pl.BlockSpec((R, H), lambda e, i, offs: (0, 0)),
    )
    return pl.pallas_call(
        _gmm_body,
        grid_spec=grid_spec,
        out_shape=jax.ShapeDtypeStruct((R, H), jnp.float32),
        compiler_params=pltpu.CompilerParams(vmem_limit_bytes=63 * 1024 * 1024),
    )(offs, xs, w1, w3, w2, sw)


def _sc_combine(ys, p0, p1):
    tpw = T // NW
    mesh = plsc.VectorSubcoreMesh(core_axis_name="c", subcore_axis_name="s")

    @functools.partial(
        pl.kernel,
        mesh=mesh,
        out_type=jax.ShapeDtypeStruct((T, H), jnp.float32),
        scratch_types=[
            pltpu.VMEM((tpw,), jnp.int32),
            pltpu.VMEM((tpw,), jnp.int32),
            pltpu.VMEM((tpw, H), jnp.float32),
            pltpu.VMEM((tpw, H), jnp.float32),
            pltpu.SemaphoreType.DMA,
        ],
    )
    def k(ys_hbm, p0_hbm, p1_hbm, out_hbm, i0_v, i1_v, a_v, b_v, sem):
        wid = lax.axis_index("s") * 2 + lax.axis_index("c")
        base = wid * tpw
        pltpu.sync_copy(p0_hbm.at[pl.ds(base, tpw)], i0_v)
        pltpu.sync_copy(p1_hbm.at[pl.ds(base, tpw)], i1_v)
        ca = pltpu.async_copy(ys_hbm.at[i0_v], a_v, sem)
        cb = pltpu.async_copy(ys_hbm.at[i1_v], b_v, sem)
        ca.wait()
        cb.wait()

        def row(i, _):
            for c in range(H // 16):
                sl = pl.ds(c * 16, 16)
                a_v[i, sl] = a_v[i, sl] + b_v[i, sl]
            return 0

        lax.fori_loop(0, tpw, row, 0)
        pltpu.sync_copy(a_v, out_hbm.at[pl.ds(base, tpw)])

    return k(ys, p0, p1)


def kernel(x, Wg, W1, W2, W3):
    b, s, h = x.shape
    x2 = x.reshape(T, H)
    pos, ends, wflat = _routing(x2, Wg)
    offs = jnp.concatenate([jnp.zeros((1,), jnp.int32), ends.reshape(E)])
    posf = pos.reshape(R)
    xs, sw = _sc_dispatch(x2, posf, wflat.reshape(R))
    ys = _gmm(offs, xs, W1, W3, W2, sw.reshape(R, 1))
    out = _sc_combine(ys, posf[:T], posf[T:])
    return out.reshape(b, s, h)
